# baseline (speedup 1.0000x reference)
import functools

import jax
import jax.numpy as jnp
from jax import lax
from jax.experimental import pallas as pl
from jax.experimental.pallas import tpu as pltpu
from jax.experimental.pallas import tpu_sc as plsc

LANES = 16
NC = 2
NS = 16
NBT = 8
NJV = 4
TILE_R, TILE_C = 8, 128


def _document_model_t(x_flat, idf, B, L):
    V = idf.shape[0]
    NQ = V // TILE_R
    QW = NQ // NJV
    SQ = 25
    n_strips = QW // SQ
    SV = SQ * TILE_R
    DIV_M, DIV_S = 5243, 17
    rows_w = B // NBT
    RC = 16
    n_rc = rows_w // RC
    n_chunks = (L + LANES - 1) // LANES
    l_pad = n_chunks * LANES
    rem_l = L - (n_chunks - 1) * LANES
    g_chunk = 104
    n_gather = RC * l_pad // g_chunk
    n_buckets = n_strips + 1
    cap = RC * l_pad * n_rc + 16 * n_buckets

    mesh = plsc.VectorSubcoreMesh(core_axis_name="c", subcore_axis_name="s")

    dnums = lax.GatherDimensionNumbers(
        offset_dims=(), collapsed_slice_dims=(0,), start_index_map=(0,))

    def lane_perm(v, idx):
        return lax.gather(v, idx[:, None], dnums, (1,),
                          mode=lax.GatherScatterMode.PROMISE_IN_BOUNDS)

    def lane_sum(v, lane):
        for s in (8, 4, 2, 1):
            v = v + lane_perm(v, lane ^ s)
        return v

    @functools.partial(
        pl.kernel,
        mesh=mesh,
        out_type=jax.ShapeDtypeStruct((V, B), jnp.float32),
        compiler_params=pltpu.CompilerParams(needs_layout_passes=False),
        scratch_types=[
            pltpu.VMEM((rows_w * l_pad,), jnp.int32),
            pltpu.VMEM((RC * l_pad,), jnp.float32),
            pltpu.VMEM((cap,), jnp.int32),
            pltpu.VMEM((cap,), jnp.float32),
            pltpu.VMEM((128,), jnp.int32),
            pltpu.VMEM((128,), jnp.int32),
            pltpu.VMEM((128,), jnp.int32),
            pltpu.VMEM((SV, TILE_C), jnp.float32),
            pltpu.SemaphoreType.DMA,
        ],
    )
    def run(x_hbm, idf_hbm, out_hbm, tok_ref, val_ref, keys, lvals,
            counts, offs, curs, sbuf, sem):
        wid = lax.axis_index("s") * NC + lax.axis_index("c")
        bt = wid % NBT
        q0 = (wid // NBT) * QW
        lane = lax.iota(jnp.int32, LANES)
        zf = jnp.zeros((LANES,), jnp.float32)
        zi = jnp.zeros((LANES,), jnp.int32)
        ones = jnp.full((LANES,), 1, jnp.int32)

        for g in range(128 // LANES):
            counts[pl.ds(LANES * g, LANES)] = zi
        def ztok(i, c):
            tok_ref[pl.ds(LANES * i, LANES)] = zi
            return c
        lax.fori_loop(0, rows_w * l_pad // LANES, ztok, 0)
        def zbuf(i, c):
            for g in range(TILE_C // LANES):
                sbuf[i, pl.ds(LANES * g, LANES)] = zf
            return c
        lax.fori_loop(0, SV, zbuf, 0)

        def stream_tokens(c):
            hs = [pltpu.async_copy(
                      x_hbm.at[pl.ds((TILE_C * bt + RC * c + r) * L, L)],
                      tok_ref.at[pl.ds((RC * c + r) * l_pad, L)], sem)
                  for r in range(RC)]
            for h in hs:
                h.wait()

        def strip_of(tok, valid):
            qrel = (tok >> 3) - q0
            m = valid & (qrel >= 0) & (qrel < QW)
            s = (qrel * DIV_M) >> DIV_S
            return jnp.where(m, s, n_strips)

        def count_chunk(c, carry):
            stream_tokens(c)
            def row_body(rr, cc):
                base = (RC * c + rr) * l_pad
                for i in range(n_chunks):
                    tok = tok_ref[pl.ds(base + LANES * i, LANES)]
                    valid = (lane < rem_l) if i == n_chunks - 1 else None
                    v = valid if valid is not None else (lane >= 0)
                    s = strip_of(tok, v)
                    plsc.addupdate_scatter(counts, [s], ones)
                return cc
            lax.fori_loop(0, RC, row_body, 0)
            return carry
        lax.fori_loop(0, n_rc, count_chunk, 0)

        run_v = zi
        fifteen = jnp.full((LANES,), 15, jnp.int32)
        for g in range(128 // LANES):
            ca = (counts[pl.ds(LANES * g, LANES)] + 15) & jnp.int32(-16)
            cs = plsc.cumsum(ca)
            off = cs - ca + run_v
            offs[pl.ds(LANES * g, LANES)] = off
            curs[pl.ds(LANES * g, LANES)] = off
            run_v = run_v + lane_perm(cs, fifteen)

        def append_chunk(c, carry):
            cbase = RC * c * l_pad
            hs = [pltpu.async_copy(
                      idf_hbm.at[tok_ref.at[pl.ds(cbase + g_chunk * j,
                                                  g_chunk)]],
                      val_ref.at[pl.ds(g_chunk * j, g_chunk)], sem)
                  for j in range(n_gather // 2)]
            for h in hs:
                h.wait()
            hs = [pltpu.async_copy(
                      idf_hbm.at[tok_ref.at[pl.ds(cbase + g_chunk * j,
                                                  g_chunk)]],
                      val_ref.at[pl.ds(g_chunk * j, g_chunk)], sem)
                  for j in range(n_gather // 2, n_gather)]
            for h in hs:
                h.wait()
            def row_body(rr, cc):
                base = rr * l_pad
                acc = zf
                for i in range(n_chunks):
                    v = val_ref[pl.ds(base + LANES * i, LANES)]
                    if i == n_chunks - 1 and rem_l != LANES:
                        v = jnp.where(lane < rem_l, v, 0.0)
                    acc = acc + v
                inv = 1.0 / lane_sum(acc, lane)
                r_glob = RC * c + rr
                for i in range(n_chunks):
                    tok = tok_ref[pl.ds(cbase + base + LANES * i, LANES)]
                    val = val_ref[pl.ds(base + LANES * i, LANES)] * inv
                    valid = (lane < rem_l) if i == n_chunks - 1 \
                        else (lane >= 0)
                    s = strip_of(tok, valid)
                    sk, sv = plsc.sort_key_val(s, lane)
                    tok_s = lane_perm(tok, sv)
                    val_s = lane_perm(val, sv)
                    prev = lane_perm(sk, (lane - 1) & 15)
                    bnd = (lane == 0) | (sk != prev)
                    firstl = plsc.cummax(jnp.where(bnd, lane, 0))
                    rank = lane - firstl
                    pos = plsc.load_gather(curs, [sk]) + rank
                    key = tok_s * 128 + r_glob
                    plsc.store_scatter(keys, [pos], key)
                    plsc.store_scatter(lvals, [pos], val_s)
                    plsc.addupdate_scatter(curs, [sk], ones)
                return cc
            lax.fori_loop(0, RC, row_body, 0)
            return carry
        lax.fori_loop(0, n_rc, append_chunk, 0)

        def load_scalar(ref, t):
            base = (t // LANES) * LANES
            vec = ref[pl.ds(base, LANES)]
            r = jnp.full((LANES,), t - base, jnp.int32)
            return lane_perm(vec, r)[0]

        def strip_body(t, carry):
            st = load_scalar(offs, t)
            en = load_scalar(curs, t)
            vbase = 8 * (q0 + SQ * t)
            def ent_body(j, cc):
                p = st + LANES * j
                m = (p + lane) < en
                k = keys[pl.ds(p, LANES)]
                val = jnp.where(m, lvals[pl.ds(p, LANES)], 0.0)
                vloc = (k >> 7) - vbase
                r128 = k & 127
                plsc.addupdate_scatter(sbuf, [vloc, r128], val, mask=m)
                return cc
            nv = (en - st + LANES - 1) // LANES
            lax.fori_loop(0, nv, ent_body, 0)
            pltpu.sync_copy(sbuf, out_hbm.at[pl.ds(vbase, SV),
                                             pl.ds(TILE_C * bt, TILE_C)])
            def clr_body(j, cc):
                p = st + LANES * j
                m = (p + lane) < en
                k = keys[pl.ds(p, LANES)]
                vloc = (k >> 7) - vbase
                r128 = k & 127
                plsc.store_scatter(sbuf, [vloc, r128], zf, mask=m)
                return cc
            lax.fori_loop(0, nv, clr_body, 0)
            return carry
        lax.fori_loop(0, n_strips, strip_body, 0)

    return run(x_flat, idf)


def kernel(x, idf):
    B, L = x.shape
    x_flat = x.astype(jnp.int32).reshape(-1)
    out_t = _document_model_t(x_flat, idf, B, L)
    return out_t.T


# speedup vs baseline: 1.0323x; 1.0323x over previous
import functools

import jax
import jax.numpy as jnp
from jax import lax
from jax.experimental import pallas as pl
from jax.experimental.pallas import tpu as pltpu
from jax.experimental.pallas import tpu_sc as plsc

LANES = 16
NC = 2
NS = 16
NBT = 8
NJV = 4
TILE_R, TILE_C = 8, 128


def _document_model_t(x_flat, idf, B, L):
    V = idf.shape[0]
    NQ = V // TILE_R
    QW = NQ // NJV
    SQ = 25
    n_strips = QW // SQ
    SV = SQ * TILE_R
    DIV_M, DIV_S = 5243, 17
    rows_w = B // NBT
    RC = 16
    n_rc = rows_w // RC
    n_chunks = (L + LANES - 1) // LANES
    l_pad = n_chunks * LANES
    rem_l = L - (n_chunks - 1) * LANES
    g_chunk = 104
    n_gather = RC * l_pad // g_chunk
    n_buckets = n_strips + 1
    cap = RC * l_pad * n_rc + 16 * n_buckets

    mesh = plsc.VectorSubcoreMesh(core_axis_name="c", subcore_axis_name="s")

    dnums = lax.GatherDimensionNumbers(
        offset_dims=(), collapsed_slice_dims=(0,), start_index_map=(0,))

    def lane_perm(v, idx):
        return lax.gather(v, idx[:, None], dnums, (1,),
                          mode=lax.GatherScatterMode.PROMISE_IN_BOUNDS)

    def lane_sum(v, lane):
        for s in (8, 4, 2, 1):
            v = v + lane_perm(v, lane ^ s)
        return v

    @functools.partial(
        pl.kernel,
        mesh=mesh,
        out_type=jax.ShapeDtypeStruct((V, B), jnp.float32),
        compiler_params=pltpu.CompilerParams(needs_layout_passes=False),
        scratch_types=[
            pltpu.VMEM((RC * l_pad,), jnp.int32),
            pltpu.VMEM((RC * l_pad,), jnp.float32),
            pltpu.VMEM((cap,), jnp.int32),
            pltpu.VMEM((cap,), jnp.float32),
            pltpu.VMEM((128,), jnp.int32),
            pltpu.VMEM((128,), jnp.int32),
            pltpu.VMEM((128,), jnp.int32),
            pltpu.VMEM((SV, TILE_C), jnp.float32),
            pltpu.VMEM((SV, TILE_C), jnp.float32),
            pltpu.SemaphoreType.DMA,
            pltpu.SemaphoreType.DMA,
            pltpu.SemaphoreType.DMA,
        ],
    )
    def run(x_hbm, idf_hbm, out_hbm, tok_ref, val_ref, keys, lvals,
            counts, offs, curs, sbuf, sbufB, sem, semSA, semSB):
        wid = lax.axis_index("s") * NC + lax.axis_index("c")
        bt = wid % NBT
        q0 = (wid // NBT) * QW
        lane = lax.iota(jnp.int32, LANES)
        zf = jnp.zeros((LANES,), jnp.float32)
        zi = jnp.zeros((LANES,), jnp.int32)
        ones = jnp.full((LANES,), 1, jnp.int32)

        for g in range(128 // LANES):
            counts[pl.ds(LANES * g, LANES)] = zi
        def ztok(i, c):
            tok_ref[pl.ds(LANES * i, LANES)] = zi
            return c
        lax.fori_loop(0, RC * l_pad // LANES, ztok, 0)
        def zbuf(i, c):
            for g in range(TILE_C // LANES):
                sbuf[i, pl.ds(LANES * g, LANES)] = zf
                sbufB[i, pl.ds(LANES * g, LANES)] = zf
            return c
        lax.fori_loop(0, SV, zbuf, 0)

        def stream_tokens(c):
            hs = [pltpu.async_copy(
                      x_hbm.at[pl.ds((TILE_C * bt + RC * c + r) * L, L)],
                      tok_ref.at[pl.ds(r * l_pad, L)], sem)
                  for r in range(RC)]
            for h in hs:
                h.wait()

        def strip_of(tok, valid):
            qrel = (tok >> 3) - q0
            m = valid & (qrel >= 0) & (qrel < QW)
            s = (qrel * DIV_M) >> DIV_S
            return jnp.where(m, s, n_strips)

        def count_chunk(c, carry):
            stream_tokens(c)
            def row_body(rr, cc):
                base = rr * l_pad
                for i in range(n_chunks):
                    tok = tok_ref[pl.ds(base + LANES * i, LANES)]
                    valid = (lane < rem_l) if i == n_chunks - 1 else None
                    v = valid if valid is not None else (lane >= 0)
                    s = strip_of(tok, v)
                    plsc.addupdate_scatter(counts, [s], ones)
                return cc
            lax.fori_loop(0, RC, row_body, 0)
            return carry
        lax.fori_loop(0, n_rc, count_chunk, 0)

        run_v = zi
        fifteen = jnp.full((LANES,), 15, jnp.int32)
        for g in range(128 // LANES):
            ca = (counts[pl.ds(LANES * g, LANES)] + 15) & jnp.int32(-16)
            cs = plsc.cumsum(ca)
            off = cs - ca + run_v
            offs[pl.ds(LANES * g, LANES)] = off
            curs[pl.ds(LANES * g, LANES)] = off
            run_v = run_v + lane_perm(cs, fifteen)

        def append_chunk(c, carry):
            stream_tokens(c)
            hs = [pltpu.async_copy(
                      idf_hbm.at[tok_ref.at[pl.ds(g_chunk * j, g_chunk)]],
                      val_ref.at[pl.ds(g_chunk * j, g_chunk)], sem)
                  for j in range(n_gather // 2)]
            for h in hs:
                h.wait()
            hs = [pltpu.async_copy(
                      idf_hbm.at[tok_ref.at[pl.ds(g_chunk * j, g_chunk)]],
                      val_ref.at[pl.ds(g_chunk * j, g_chunk)], sem)
                  for j in range(n_gather // 2, n_gather)]
            for h in hs:
                h.wait()
            def row_body(rr, cc):
                base = rr * l_pad
                acc = zf
                for i in range(n_chunks):
                    v = val_ref[pl.ds(base + LANES * i, LANES)]
                    if i == n_chunks - 1 and rem_l != LANES:
                        v = jnp.where(lane < rem_l, v, 0.0)
                    acc = acc + v
                inv = 1.0 / lane_sum(acc, lane)
                r_glob = RC * c + rr
                for i in range(n_chunks):
                    tok = tok_ref[pl.ds(base + LANES * i, LANES)]
                    val = val_ref[pl.ds(base + LANES * i, LANES)] * inv
                    valid = (lane < rem_l) if i == n_chunks - 1 \
                        else (lane >= 0)
                    s = strip_of(tok, valid)
                    sk, sv = plsc.sort_key_val(s, lane)
                    tok_s = lane_perm(tok, sv)
                    val_s = lane_perm(val, sv)
                    prev = lane_perm(sk, (lane - 1) & 15)
                    bnd = (lane == 0) | (sk != prev)
                    firstl = plsc.cummax(jnp.where(bnd, lane, 0))
                    rank = lane - firstl
                    pos = plsc.load_gather(curs, [sk]) + rank
                    key = tok_s * 128 + r_glob
                    plsc.store_scatter(keys, [pos], key)
                    plsc.store_scatter(lvals, [pos], val_s)
                    plsc.addupdate_scatter(curs, [sk], ones)
                return cc
            lax.fori_loop(0, RC, row_body, 0)
            return carry
        lax.fori_loop(0, n_rc, append_chunk, 0)

        def load_scalar(ref, t):
            base = (t // LANES) * LANES
            vec = ref[pl.ds(base, LANES)]
            r = jnp.full((LANES,), t - base, jnp.int32)
            return lane_perm(vec, r)[0]

        def scatter_strip(buf, t, st, en):
            vbase = 8 * (q0 + SQ * t)
            def ent_body(j, cc):
                p = st + LANES * j
                m = (p + lane) < en
                k = keys[pl.ds(p, LANES)]
                val = jnp.where(m, lvals[pl.ds(p, LANES)], 0.0)
                plsc.addupdate_scatter(buf, [(k >> 7) - vbase, k & 127],
                                       val, mask=m)
                return cc
            lax.fori_loop(0, (en - st + LANES - 1) // LANES, ent_body, 0)

        def clear_strip(buf, t, st, en):
            vbase = 8 * (q0 + SQ * t)
            def clr_body(j, cc):
                p = st + LANES * j
                m = (p + lane) < en
                k = keys[pl.ds(p, LANES)]
                plsc.store_scatter(buf, [(k >> 7) - vbase, k & 127], zf,
                                   mask=m)
                return cc
            lax.fori_loop(0, (en - st + LANES - 1) // LANES, clr_body, 0)

        def out_view(t):
            return out_hbm.at[pl.ds(8 * (q0 + SQ * t), SV),
                              pl.ds(TILE_C * bt, TILE_C)]

        def strip_pair(k, carry):
            t0 = 2 * k
            t1 = t0 + 1
            st0 = load_scalar(offs, t0)
            en0 = load_scalar(curs, t0)
            scatter_strip(sbuf, t0, st0, en0)
            h0 = pltpu.async_copy(sbuf, out_view(t0), semSA)
            st1 = load_scalar(offs, t1)
            en1 = load_scalar(curs, t1)
            scatter_strip(sbufB, t1, st1, en1)
            h1 = pltpu.async_copy(sbufB, out_view(t1), semSB)
            h0.wait()
            clear_strip(sbuf, t0, st0, en0)
            h1.wait()
            clear_strip(sbufB, t1, st1, en1)
            return carry
        lax.fori_loop(0, n_strips // 2, strip_pair, 0)
        tl = jnp.int32(n_strips - 1)
        stl = load_scalar(offs, tl)
        enl = load_scalar(curs, tl)
        scatter_strip(sbuf, tl, stl, enl)
        pltpu.async_copy(sbuf, out_view(tl), semSA).wait()

    return run(x_flat, idf)


def kernel(x, idf):
    B, L = x.shape
    x_flat = x.astype(jnp.int32).reshape(-1)
    out_t = _document_model_t(x_flat, idf, B, L)
    return out_t.T
